# Initial kernel scaffold; baseline (speedup 1.0000x reference)
import jax
import jax.numpy as jnp
from jax.experimental import pallas as pl


def kernel(x_bd):
    raise NotImplementedError("write your pallas kernel here")


# speedup vs baseline: 2.1371x; 2.1371x over previous
import jax
import jax.numpy as jnp
from jax.experimental import pallas as pl
from jax.experimental.pallas import tpu as pltpu

_MARK_AX = 0
_OP_AND = 1
_OP_OR = 2
_OP_XOR = 3
_ALU_LO = 4
_ALU_HI = 20
_AX_CARRY_LO = 36
_AX_CARRY_HI = 52
_OUTPUT_LO = 68
_OUTPUT_HI = 84
_DIM = 100

_ROWS_PER_BLOCK = 2048


def _body(x_ref, o_ref):
    x = x_ref[...]
    r = x.shape[0]

    is_mark = x[:, _MARK_AX:_MARK_AX + 1] >= 0.5
    is_and = x[:, _OP_AND:_OP_AND + 1] > 0.5
    is_or = x[:, _OP_OR:_OP_OR + 1] > 0.5
    is_xor = x[:, _OP_XOR:_OP_XOR + 1] > 0.5
    active = is_mark & (is_and | is_or | is_xor)

    iota16 = jax.lax.broadcasted_iota(jnp.int32, (r, 16), 1)

    def first_set(lo):
        m = x[:, lo:lo + 16] > 0.5
        idx = jnp.min(jnp.where(m, iota16, 16), axis=1, keepdims=True)
        return jnp.where(idx < 16, idx, 0)

    a_lo = first_set(_ALU_LO)
    a_hi = first_set(_ALU_HI)
    b_lo = first_set(_AX_CARRY_LO)
    b_hi = first_set(_AX_CARRY_HI)

    def op(a, b):
        return jnp.where(is_and, a & b, jnp.where(is_or, a | b, a ^ b))

    r_lo = op(a_lo, b_lo)
    r_hi = op(a_hi, b_hi)

    iota100 = jax.lax.broadcasted_iota(jnp.int32, (r, _DIM), 1)
    add = jnp.where(active, jnp.float32(2.0), jnp.float32(0.0))
    delta = (jnp.where(iota100 == _OUTPUT_LO + r_lo, add, 0.0)
             + jnp.where(iota100 == _OUTPUT_HI + r_hi, add, 0.0))
    o_ref[...] = x + delta


def kernel(x_bd):
    b, s, d = x_bd.shape
    n = b * s
    flat = x_bd.reshape(n, d)
    rows = _ROWS_PER_BLOCK
    out = pl.pallas_call(
        _body,
        grid=(n // rows,),
        in_specs=[pl.BlockSpec((rows, d), lambda i: (i, 0))],
        out_specs=pl.BlockSpec((rows, d), lambda i: (i, 0)),
        out_shape=jax.ShapeDtypeStruct((n, d), x_bd.dtype),
        compiler_params=pltpu.CompilerParams(
            dimension_semantics=("arbitrary",),
        ),
    )(flat)
    return out.reshape(b, s, d)


# speedup vs baseline: 2.5376x; 1.1874x over previous
import functools

import jax
import jax.numpy as jnp
from jax import lax
from jax.experimental import pallas as pl
from jax.experimental.pallas import tpu as pltpu
from jax.experimental.pallas import tpu_sc as plsc

_MARK_AX = 0
_OP_AND = 1
_OP_OR = 2
_OP_XOR = 3
_ALU_LO = 4
_ALU_HI = 20
_AX_CARRY_LO = 36
_AX_CARRY_HI = 52
_OUTPUT_LO = 68
_OUTPUT_HI = 84
_DIM = 100

_NW = 32
_ROWS_PER_CHUNK = 256
_CHUNK_WORDS = _ROWS_PER_CHUNK * _DIM


def _make_sc_kernel(n_rows):
    rows_per_w = n_rows // _NW
    chunks = rows_per_w // _ROWS_PER_CHUNK
    groups = _ROWS_PER_CHUNK // 16
    mesh = plsc.VectorSubcoreMesh(core_axis_name="c", subcore_axis_name="s")

    @functools.partial(
        pl.kernel,
        mesh=mesh,
        out_type=jax.ShapeDtypeStruct((n_rows * _DIM,), jnp.float32),
        scratch_types=[
            pltpu.VMEM((_CHUNK_WORDS,), jnp.float32),
            pltpu.SemaphoreType.DMA,
        ],
        compiler_params=pltpu.CompilerParams(needs_layout_passes=False),
    )
    def sc_kernel(x_hbm, out_hbm, buf, sem):
        wid = lax.axis_index("s") * 2 + lax.axis_index("c")
        w_base = wid * (rows_per_w * _DIM)
        lane = lax.iota(jnp.int32, 16)
        row_off0 = lane * _DIM

        def do_group(j, _):
            row_off = row_off0 + j * (16 * _DIM)

            def col(c):
                return plsc.load_gather(buf, [row_off + c])

            is_mark = col(_MARK_AX) >= 0.5
            is_and = col(_OP_AND) > 0.5
            is_or = col(_OP_OR) > 0.5
            is_xor = col(_OP_XOR) > 0.5
            active = is_mark & (is_and | is_or | is_xor)

            def first_set(w):
                res = jnp.full((16,), 16, jnp.int32)
                for c in range(15, -1, -1):
                    m = col(w + c) > 0.5
                    res = jnp.where(m, jnp.int32(c), res)
                return jnp.where(res < 16, res, 0)

            a_lo = first_set(_ALU_LO)
            a_hi = first_set(_ALU_HI)
            b_lo = first_set(_AX_CARRY_LO)
            b_hi = first_set(_AX_CARRY_HI)

            def op(a, b):
                return jnp.where(is_and, a & b,
                                 jnp.where(is_or, a | b, a ^ b))

            r_lo = op(a_lo, b_lo)
            r_hi = op(a_hi, b_hi)

            add = jnp.full((16,), 2.0, jnp.float32)
            plsc.addupdate_scatter(buf, [row_off + (_OUTPUT_LO + r_lo)],
                                   add, mask=active)
            plsc.addupdate_scatter(buf, [row_off + (_OUTPUT_HI + r_hi)],
                                   add, mask=active)
            return 0

        def do_chunk(g, _):
            start = w_base + g * _CHUNK_WORDS
            pltpu.sync_copy(x_hbm.at[pl.ds(start, _CHUNK_WORDS)], buf)
            lax.fori_loop(0, groups, do_group, 0)
            pltpu.sync_copy(buf, out_hbm.at[pl.ds(start, _CHUNK_WORDS)])
            return 0

        lax.fori_loop(0, chunks, do_chunk, 0)

    return sc_kernel


def kernel(x_bd):
    b, s, d = x_bd.shape
    n = b * s
    flat = x_bd.reshape(n * d)
    out = _make_sc_kernel(n)(flat)
    return out.reshape(b, s, d)


# speedup vs baseline: 3.4913x; 1.3758x over previous
import functools

import jax
import jax.numpy as jnp
from jax import lax
from jax.experimental import pallas as pl
from jax.experimental.pallas import tpu as pltpu
from jax.experimental.pallas import tpu_sc as plsc

_MARK_AX = 0
_OP_AND = 1
_OP_OR = 2
_OP_XOR = 3
_ALU_LO = 4
_ALU_HI = 20
_AX_CARRY_LO = 36
_AX_CARRY_HI = 52
_OUTPUT_LO = 68
_OUTPUT_HI = 84
_DIM = 100

_NW = 32
_ROWS_PER_CHUNK = 256


def _make_sc_kernel(n_rows):
    rows_per_w = n_rows // _NW
    chunks = rows_per_w // _ROWS_PER_CHUNK
    groups = _ROWS_PER_CHUNK // 16
    mesh = plsc.VectorSubcoreMesh(core_axis_name="c", subcore_axis_name="s")

    @functools.partial(
        pl.kernel,
        mesh=mesh,
        out_type=jax.ShapeDtypeStruct((n_rows, _DIM), jnp.float32),
        scratch_types=[
            pltpu.VMEM((_ROWS_PER_CHUNK, _DIM), jnp.float32),
            pltpu.SemaphoreType.DMA,
        ],
        compiler_params=pltpu.CompilerParams(
            needs_layout_passes=False,
            use_tc_tiling_on_sc=True,
        ),
    )
    def sc_kernel(x_hbm, out_hbm, buf, sem):
        wid = lax.axis_index("s") * 2 + lax.axis_index("c")
        w_base = wid * rows_per_w
        lane = lax.iota(jnp.int32, 16)

        def do_group(j, _):
            rows = j * 16 + lane

            def col(c):
                return plsc.load_gather(
                    buf, [rows, jnp.full((16,), c, jnp.int32)])

            is_mark = col(_MARK_AX) >= 0.5
            is_and = col(_OP_AND) > 0.5
            is_or = col(_OP_OR) > 0.5
            is_xor = col(_OP_XOR) > 0.5
            active = is_mark & (is_and | is_or | is_xor)

            def first_set(w):
                res = jnp.full((16,), 16, jnp.int32)
                for c in range(15, -1, -1):
                    m = col(w + c) > 0.5
                    res = jnp.where(m, jnp.int32(c), res)
                return jnp.where(res < 16, res, 0)

            a_lo = first_set(_ALU_LO)
            a_hi = first_set(_ALU_HI)
            b_lo = first_set(_AX_CARRY_LO)
            b_hi = first_set(_AX_CARRY_HI)

            def op(a, b):
                return jnp.where(is_and, a & b,
                                 jnp.where(is_or, a | b, a ^ b))

            r_lo = op(a_lo, b_lo)
            r_hi = op(a_hi, b_hi)

            add = jnp.full((16,), 2.0, jnp.float32)
            plsc.addupdate_scatter(buf, [rows, _OUTPUT_LO + r_lo],
                                   add, mask=active)
            plsc.addupdate_scatter(buf, [rows, _OUTPUT_HI + r_hi],
                                   add, mask=active)
            return 0

        def do_chunk(g, _):
            start = w_base + g * _ROWS_PER_CHUNK
            pltpu.sync_copy(x_hbm.at[pl.ds(start, _ROWS_PER_CHUNK), :], buf)
            lax.fori_loop(0, groups, do_group, 0)
            pltpu.sync_copy(buf, out_hbm.at[pl.ds(start, _ROWS_PER_CHUNK), :])
            return 0

        lax.fori_loop(0, chunks, do_chunk, 0)

    return sc_kernel


def kernel(x_bd):
    b, s, d = x_bd.shape
    n = b * s
    flat = x_bd.reshape(n, d)
    out = _make_sc_kernel(n)(flat)
    return out.reshape(b, s, d)
